# Initial kernel scaffold; baseline (speedup 1.0000x reference)
import jax
import jax.numpy as jnp
from jax.experimental import pallas as pl


def kernel(image_stack, node_categories, edge_categories, edge_connections, graph_idx_of_node, graph_idx_of_edge, node_table, edge_table, Wconv, bconv, Wfc, bfc, W_e0, b_e0, W_v0, b_v0, W_g0, b_g0, W_e1, b_e1, W_v1, b_v1, W_g1, b_g1, W_e2, b_e2, W_v2, b_v2, W_g2, b_g2):
    raise NotImplementedError("write your pallas kernel here")


# speedup vs baseline: 2.0273x; 2.0273x over previous
import functools

import jax
import jax.numpy as jnp
from jax import lax
from jax.experimental import pallas as pl
from jax.experimental.pallas import tpu as pltpu
from jax.experimental.pallas import tpu_sc as plsc

N_NODES = 10000
N_EDGES = 320000
NG = 16
D = 128
NC, NS, L = 2, 16, 16
NW = NC * NS
EPW = N_EDGES // NW
CH = 80
NCHE = EPW // CH
NPAD = 10240

_f32 = jnp.float32


def _cnn_pool_body(p_ref, wf_ref, bc_ref, out_ref):
    y = jnp.dot(p_ref[...], wf_ref[...], preferred_element_type=_f32)
    y = jnp.maximum(y + bc_ref[...], 0.0)
    s = jnp.sum(y, axis=0) * (1.0 / 12544.0)
    out_ref[...] = s.reshape(1, 1, 16)


def _cnn_pool(patches, wf, bc):
    return pl.pallas_call(
        _cnn_pool_body,
        grid=(NG,),
        in_specs=[
            pl.BlockSpec((12544, 32), lambda g: (g, 0)),
            pl.BlockSpec((32, 16), lambda g: (0, 0)),
            pl.BlockSpec((1, 16), lambda g: (0, 0)),
        ],
        out_specs=pl.BlockSpec((1, 1, 16), lambda g: (g, 0, 0)),
        out_shape=jax.ShapeDtypeStruct((NG, 1, 16), _f32),
    )(patches, wf, bc)


def _prep0_body(pool_ref, wfc_ref, bfc_ref, et_ref, nt_ref, ae_ref, ag_ref,
                vg_ref, as_ref, ad_ref, be_ref, g0_ref, tc_ref, tns_ref,
                tnd_ref, tgv_ref):
    g0 = jnp.dot(pool_ref[...], wfc_ref[...], preferred_element_type=_f32) + bfc_ref[...]
    g0_ref[...] = g0
    te = jnp.dot(et_ref[...], ae_ref[...], preferred_element_type=_f32)
    tg = jnp.dot(g0, ag_ref[...], preferred_element_type=_f32)
    tc = te[:, None, :] + tg[None, :, :]
    tc_ref[...] = tc.reshape(256, D) + be_ref[...]
    tns_ref[...] = jnp.dot(nt_ref[...], as_ref[...], preferred_element_type=_f32)
    tnd_ref[...] = jnp.dot(nt_ref[...], ad_ref[...], preferred_element_type=_f32)
    tgv_ref[...] = jnp.dot(g0, vg_ref[...], preferred_element_type=_f32)


def _prep0(pooled, wfc, bfc, et, nt, ae, ag, vg, a_s, a_d, be):
    return pl.pallas_call(
        _prep0_body,
        out_shape=(
            jax.ShapeDtypeStruct((NG, D), _f32),
            jax.ShapeDtypeStruct((256, D), _f32),
            jax.ShapeDtypeStruct((32, D), _f32),
            jax.ShapeDtypeStruct((32, D), _f32),
            jax.ShapeDtypeStruct((NG, D), _f32),
        ),
    )(pooled, wfc, bfc, et, nt, ae, ag, vg, a_s, a_d, be)


def _gprep_body(g_ref, ag_ref, vg_ref, tge_ref, tgv_ref):
    tge_ref[...] = jnp.dot(g_ref[...], ag_ref[...], preferred_element_type=_f32)
    tgv_ref[...] = jnp.dot(g_ref[...], vg_ref[...], preferred_element_type=_f32)


def _gprep(g, ag, vg):
    return pl.pallas_call(
        _gprep_body,
        out_shape=(jax.ShapeDtypeStruct((NG, D), _f32),
                   jax.ShapeDtypeStruct((NG, D), _f32)),
    )(g, ag, vg)


def _ea_body(e_ref, gie_ref, w_ref, b_ref, out_ref):
    gv = gie_ref[0, 0, :]
    oh = (gv[:, None] == lax.broadcasted_iota(jnp.int32, (1000, NG), 1)).astype(_f32)
    x = jnp.concatenate([e_ref[...], oh], axis=1)
    out_ref[...] = jnp.dot(x, w_ref[...], preferred_element_type=_f32) + b_ref[...]


def _ea(e_prev, gie3, wcat, be):
    return pl.pallas_call(
        _ea_body,
        grid=(N_EDGES // 1000,),
        in_specs=[
            pl.BlockSpec((1000, D), lambda i: (i, 0)),
            pl.BlockSpec((1, 1, 1000), lambda i: (i, 0, 0)),
            pl.BlockSpec((D + NG, D), lambda i: (0, 0)),
            pl.BlockSpec((1, D), lambda i: (0, 0)),
        ],
        out_specs=pl.BlockSpec((1000, D), lambda i: (i, 0)),
        out_shape=jax.ShapeDtypeStruct((N_EDGES, D), _f32),
    )(e_prev, gie3, wcat, be)


def _nprep_body(n_ref, as_ref, ad_ref, ps_ref, pd_ref):
    ps_ref[...] = jnp.dot(n_ref[...], as_ref[...], preferred_element_type=_f32)
    pd_ref[...] = jnp.dot(n_ref[...], ad_ref[...], preferred_element_type=_f32)


def _nprep(n, a_s, a_d):
    return pl.pallas_call(
        _nprep_body,
        grid=(N_NODES // 1000,),
        in_specs=[
            pl.BlockSpec((1000, D), lambda i: (i, 0)),
            pl.BlockSpec((D, D), lambda i: (0, 0)),
            pl.BlockSpec((D, D), lambda i: (0, 0)),
        ],
        out_specs=(pl.BlockSpec((1000, D), lambda i: (i, 0)),
                   pl.BlockSpec((1000, D), lambda i: (i, 0))),
        out_shape=(jax.ShapeDtypeStruct((N_NODES, D), _f32),
                   jax.ShapeDtypeStruct((N_NODES, D), _f32)),
    )(n, a_s, a_d)


def _nv_body(n_ref, a0_ref, a1_ref, gin_ref, vn_ref, ve_ref, tgv_ref, bv_ref,
             out_ref, agg_ref):
    gv = gin_ref[0, 0, :]
    oh = (gv[:, None] == lax.broadcasted_iota(jnp.int32, (1000, NG), 1)).astype(_f32)
    x = jnp.dot(n_ref[...], vn_ref[...], preferred_element_type=_f32)
    x += jnp.dot(a0_ref[...] + a1_ref[...], ve_ref[...], preferred_element_type=_f32)
    x += jnp.dot(oh, tgv_ref[...], preferred_element_type=_f32)
    nn = jnp.maximum(x + bv_ref[...], 0.0)
    out_ref[...] = nn

    @pl.when(pl.program_id(0) == 0)
    def _():
        agg_ref[...] = jnp.zeros_like(agg_ref)

    agg_ref[...] += lax.dot_general(oh, nn, (((0,), (0,)), ((), ())),
                                    preferred_element_type=_f32)


def _nv(n, a0, a1, gin3, vn, ve, tgv, bv):
    return pl.pallas_call(
        _nv_body,
        grid=(N_NODES // 1000,),
        in_specs=[
            pl.BlockSpec((1000, D), lambda i: (i, 0)),
            pl.BlockSpec((1000, D), lambda i: (i, 0)),
            pl.BlockSpec((1000, D), lambda i: (i, 0)),
            pl.BlockSpec((1, 1, 1000), lambda i: (i, 0, 0)),
            pl.BlockSpec((D, D), lambda i: (0, 0)),
            pl.BlockSpec((D, D), lambda i: (0, 0)),
            pl.BlockSpec((NG, D), lambda i: (0, 0)),
            pl.BlockSpec((1, D), lambda i: (0, 0)),
        ],
        out_specs=(pl.BlockSpec((1000, D), lambda i: (i, 0)),
                   pl.BlockSpec((NG, D), lambda i: (0, 0))),
        out_shape=(jax.ShapeDtypeStruct((N_NODES, D), _f32),
                   jax.ShapeDtypeStruct((NG, D), _f32)),
    )(n, a0, a1, gin3, vn, ve, tgv, bv)


def _gv_body(g_ref, an_ref, e0_ref, e1_ref, wg_ref, bg_ref, out_ref):
    x = jnp.concatenate([g_ref[...], an_ref[...], e0_ref[...] + e1_ref[...]],
                        axis=1)
    y = jnp.dot(x, wg_ref[...], preferred_element_type=_f32) + bg_ref[...]
    out_ref[...] = jnp.maximum(y, 0.0)


def _gv(g, aggn, ge0, ge1, wg, bg):
    return pl.pallas_call(
        _gv_body,
        out_shape=jax.ShapeDtypeStruct((NG, D), _f32),
    )(g, aggn, ge0, ge1, wg, bg)


def _sc_mesh():
    return plsc.VectorSubcoreMesh(core_axis_name="c", subcore_axis_name="s",
                                  num_cores=NC)


def _make_gather(B, T):
    nch = B // (NW * CH)

    @functools.partial(
        pl.kernel,
        out_type=jax.ShapeDtypeStruct((B, D), _f32),
        mesh=_sc_mesh(),
        scratch_types=[
            pltpu.VMEM((CH,), jnp.int32),
            pltpu.VMEM((CH, D), _f32),
            pltpu.SemaphoreType.DMA,
        ],
    )
    def k(table_h, idx_h, out_h, idx_v, rows_v, sem):
        wid = lax.axis_index("s") * NC + lax.axis_index("c")
        bpw = nch * CH

        def body(i, carry):
            base = wid * bpw + i * CH
            pltpu.sync_copy(idx_h.at[pl.ds(base, CH)], idx_v)
            pltpu.async_copy(table_h.at[idx_v], rows_v, sem).wait()
            pltpu.sync_copy(rows_v, out_h.at[pl.ds(base, CH)])
            return carry

        lax.fori_loop(0, nch, body, 0)

    return k


def _make_edge_stage():
    @functools.partial(
        pl.kernel,
        out_type=(
            jax.ShapeDtypeStruct((N_EDGES, D), _f32),
            jax.ShapeDtypeStruct((NC, N_NODES, D), _f32),
            jax.ShapeDtypeStruct((NC, NG, D), _f32),
        ),
        mesh=_sc_mesh(),
        scratch_types=[
            pltpu.VMEM((CH, D), _f32),
            pltpu.VMEM((CH, D), _f32),
            pltpu.VMEM((CH, D), _f32),
            pltpu.VMEM((CH, D), _f32),
            pltpu.VMEM((CH,), jnp.int32),
            pltpu.VMEM((2, CH), jnp.int32),
            pltpu.VMEM_SHARED((N_NODES, D), _f32),
            pltpu.VMEM_SHARED((NG, D), _f32),
            pltpu.SemaphoreType.DMA,
            pltpu.SemaphoreType.DMA,
        ],
    )
    def k(ea_h, src_h, dst_h, gie_h, ps_h, pd_h, zeros_h,
          eout_h, aggn_h, aggg_h,
          ea_v, ps_v, pd_v, en_v, sidx_v, scidx_v, aggn_s, aggg_s, sem1, sem2):
        c = lax.axis_index("c")
        s = lax.axis_index("s")
        wid = s * NC + c

        @pl.when(s == 0)
        def _init():
            pltpu.sync_copy(zeros_h, aggn_s)
            pltpu.sync_copy(zeros_h.at[pl.ds(0, NG)], aggg_s)

        plsc.subcore_barrier()

        def body(i, carry):
            base = wid * EPW + i * CH
            pltpu.sync_copy(ea_h.at[pl.ds(base, CH)], ea_v)
            pltpu.sync_copy(src_h.at[pl.ds(base, CH)], sidx_v)
            pltpu.sync_copy(dst_h.at[pl.ds(base, CH)], scidx_v.at[0])
            pltpu.sync_copy(gie_h.at[pl.ds(base, CH)], scidx_v.at[1])
            cp1 = pltpu.async_copy(ps_h.at[sidx_v], ps_v, sem1)
            cp2 = pltpu.async_copy(pd_h.at[scidx_v.at[0]], pd_v, sem2)
            cp1.wait()
            cp2.wait()

            def row(r, rc):
                for cb in range(D // L):
                    sl = (r, pl.ds(cb * L, L))
                    en_v[sl] = jnp.maximum(ea_v[sl] + ps_v[sl] + pd_v[sl], 0.0)
                return rc

            lax.fori_loop(0, CH, row, 0)
            pltpu.sync_copy(en_v, eout_h.at[pl.ds(base, CH)])
            pltpu.sync_copy(en_v, aggn_s.at[scidx_v.at[0]], add=True)
            pltpu.sync_copy(en_v, aggg_s.at[scidx_v.at[1]], add=True)
            return carry

        lax.fori_loop(0, NCHE, body, 0)
        plsc.subcore_barrier()

        @pl.when(s < 10)
        def _out_n():
            pltpu.sync_copy(aggn_s.at[pl.ds(s * 1000, 1000)],
                            aggn_h.at[c, pl.ds(s * 1000, 1000)])

        @pl.when(s == 0)
        def _out_g():
            pltpu.sync_copy(aggg_s, aggg_h.at[c])

    return k


def _build_patches(image_stack):
    x = jnp.pad(image_stack, ((0, 0), (0, 0), (0, 1), (0, 1)))
    sl = []
    for di in range(3):
        for dj in range(3):
            sl.append(lax.slice(x, (0, 0, di, dj), (NG, 3, di + 223, dj + 223),
                                (1, 1, 2, 2)))
    p = jnp.stack(sl, axis=-1)
    p = p.transpose(0, 2, 3, 1, 4).reshape(NG, 12544, 27)
    p = jnp.pad(p, ((0, 0), (0, 0), (0, 5)))
    return p.reshape(NG * 12544, 32)


def _pad_w(w, rows, cols):
    return jnp.pad(w, ((0, rows - w.shape[0]), (0, cols - w.shape[1])))


def kernel(image_stack, node_categories, edge_categories, edge_connections,
           graph_idx_of_node, graph_idx_of_edge, node_table, edge_table,
           Wconv, bconv, Wfc, bfc, W_e0, b_e0, W_v0, b_v0, W_g0, b_g0,
           W_e1, b_e1, W_v1, b_v1, W_g1, b_g1, W_e2, b_e2, W_v2, b_v2,
           W_g2, b_g2):
    src = edge_connections[0].astype(jnp.int32)
    dst = edge_connections[1].astype(jnp.int32)
    gie = graph_idx_of_edge.astype(jnp.int32)
    gin = graph_idx_of_node.astype(jnp.int32)
    ec = edge_categories.astype(jnp.int32)
    nc_ = node_categories.astype(jnp.int32)

    gie3 = gie.reshape(N_EDGES // 1000, 1, 1000)
    gin3 = gin.reshape(N_NODES // 1000, 1, 1000)
    zeros_big = jnp.zeros((N_NODES, D), _f32)

    W_e2p = _pad_w(W_e2, 512, D)
    b_e2p = jnp.pad(b_e2, (0, D - 1))
    V_n2 = _pad_w(W_v2[:D], D, D)
    V_e2 = _pad_w(W_v2[D:D + 1], D, D)
    V_g2 = _pad_w(W_v2[D + 1:], D, D)
    bv2p = jnp.pad(b_v2, (0, D - 1))
    G_g2 = W_g2[:D]
    G_n2 = _pad_w(W_g2[D:D + 1], D, D)
    G_e2 = _pad_w(W_g2[D + 1:D + 2], D, D)

    layers = []
    for (We, be, Wv, bv, Wg, bg) in ((W_e0, b_e0, W_v0, b_v0, W_g0, b_g0),
                                     (W_e1, b_e1, W_v1, b_v1, W_g1, b_g1)):
        layers.append(dict(
            A_e=We[:D], A_s=We[D:2 * D], A_d=We[2 * D:3 * D], A_g=We[3 * D:],
            be=be.reshape(1, D),
            V_n=Wv[:D], V_e=Wv[D:2 * D], V_g=Wv[2 * D:], bv=bv.reshape(1, D),
            Wgcat=jnp.concatenate([Wg[:D], Wg[D:2 * D], Wg[2 * D:]], axis=0),
            bg=bg.reshape(1, D)))
    layers.append(dict(
        A_e=W_e2p[:D], A_s=W_e2p[D:2 * D], A_d=W_e2p[2 * D:3 * D],
        A_g=W_e2p[3 * D:], be=b_e2p.reshape(1, D),
        V_n=V_n2, V_e=V_e2, V_g=V_g2, bv=bv2p.reshape(1, D),
        Wgcat=jnp.concatenate([G_g2, G_n2, G_e2], axis=0),
        bg=b_g2.reshape(1, D)))

    patches = _build_patches(image_stack)
    wf = jnp.pad(Wconv.reshape(16, 27).T, ((0, 5), (0, 0)))
    pooled = _cnn_pool(patches, wf, bconv.reshape(1, 16)).reshape(NG, 16)
    g, tcomb, tns, tnd, tgv0 = _prep0(
        pooled, Wfc, bfc.reshape(1, D), edge_table, node_table,
        layers[0]["A_e"], layers[0]["A_g"], layers[0]["V_g"],
        layers[0]["A_s"], layers[0]["A_d"], layers[0]["be"])

    gather_e = _make_gather(N_EDGES, 256)
    gather_n = _make_gather(NPAD, 32)
    edge_stage = _make_edge_stage()

    idx0 = ec * NG + gie
    ncp = jnp.pad(nc_, (0, NPAD - N_NODES))
    ea0 = gather_e(tcomb, idx0)
    ps = gather_n(tns, ncp)
    pd = gather_n(tnd, ncp)
    n = gather_n(node_table.astype(_f32), ncp)[:N_NODES]

    e_prev = None
    for i, lay in enumerate(layers):
        if i == 0:
            ea = ea0
            tgv = tgv0
        else:
            tge, tgv = _gprep(g, lay["A_g"], lay["V_g"])
            wcat = jnp.concatenate([lay["A_e"], tge], axis=0)
            ea = _ea(e_prev, gie3, wcat, lay["be"])
            ps, pd = _nprep(n, lay["A_s"], lay["A_d"])
        e_new, aggn, aggg = edge_stage(ea, src, dst, gie, ps, pd, zeros_big)
        n_new, agg_gn = _nv(n, aggn[0], aggn[1], gin3, lay["V_n"], lay["V_e"],
                            tgv, lay["bv"])
        g = _gv(g, agg_gn, aggg[0], aggg[1], lay["Wgcat"], lay["bg"])
        n, e_prev = n_new, e_new

    return (g, n[:, :1], e_prev[:, :1])


# speedup vs baseline: 2.3173x; 1.1430x over previous
import functools

import jax
import jax.numpy as jnp
from jax import lax
from jax.experimental import pallas as pl
from jax.experimental.pallas import tpu as pltpu
from jax.experimental.pallas import tpu_sc as plsc

N_NODES = 10000
N_EDGES = 320000
NG = 16
D = 128
NC, NS, L = 2, 16, 16
NW = NC * NS
EPW = N_EDGES // NW
CH = 80
NCHE = EPW // CH
NPAD = 10240

_f32 = jnp.float32


def _cnn_pool_body(p_ref, wf_ref, bc_ref, out_ref):
    y = jnp.dot(p_ref[...], wf_ref[...], preferred_element_type=_f32)
    y = jnp.maximum(y + bc_ref[...], 0.0)
    s = jnp.sum(y, axis=0) * (1.0 / 12544.0)
    out_ref[...] = s.reshape(1, 1, 16)


def _cnn_pool(patches, wf, bc):
    return pl.pallas_call(
        _cnn_pool_body,
        grid=(NG,),
        in_specs=[
            pl.BlockSpec((12544, 32), lambda g: (g, 0)),
            pl.BlockSpec((32, 16), lambda g: (0, 0)),
            pl.BlockSpec((1, 16), lambda g: (0, 0)),
        ],
        out_specs=pl.BlockSpec((1, 1, 16), lambda g: (g, 0, 0)),
        out_shape=jax.ShapeDtypeStruct((NG, 1, 16), _f32),
    )(patches, wf, bc)


def _prep0_body(pool_ref, wfc_ref, bfc_ref, et_ref, nt_ref, ae_ref, ag_ref,
                vg_ref, as_ref, ad_ref, be_ref, g0_ref, tc_ref, tns_ref,
                tnd_ref, tgv_ref):
    g0 = jnp.dot(pool_ref[...], wfc_ref[...], preferred_element_type=_f32) + bfc_ref[...]
    g0_ref[...] = g0
    te = jnp.dot(et_ref[...], ae_ref[...], preferred_element_type=_f32)
    tg = jnp.dot(g0, ag_ref[...], preferred_element_type=_f32)
    tc = te[:, None, :] + tg[None, :, :]
    tc_ref[...] = tc.reshape(256, D) + be_ref[...]
    tns_ref[...] = jnp.dot(nt_ref[...], as_ref[...], preferred_element_type=_f32)
    tnd_ref[...] = jnp.dot(nt_ref[...], ad_ref[...], preferred_element_type=_f32)
    tgv_ref[...] = jnp.dot(g0, vg_ref[...], preferred_element_type=_f32)


def _prep0(pooled, wfc, bfc, et, nt, ae, ag, vg, a_s, a_d, be):
    return pl.pallas_call(
        _prep0_body,
        out_shape=(
            jax.ShapeDtypeStruct((NG, D), _f32),
            jax.ShapeDtypeStruct((256, D), _f32),
            jax.ShapeDtypeStruct((32, D), _f32),
            jax.ShapeDtypeStruct((32, D), _f32),
            jax.ShapeDtypeStruct((NG, D), _f32),
        ),
    )(pooled, wfc, bfc, et, nt, ae, ag, vg, a_s, a_d, be)


def _gprep_body(g_ref, ag_ref, vg_ref, tge_ref, tgv_ref):
    tge_ref[...] = jnp.dot(g_ref[...], ag_ref[...], preferred_element_type=_f32)
    tgv_ref[...] = jnp.dot(g_ref[...], vg_ref[...], preferred_element_type=_f32)


def _gprep(g, ag, vg):
    return pl.pallas_call(
        _gprep_body,
        out_shape=(jax.ShapeDtypeStruct((NG, D), _f32),
                   jax.ShapeDtypeStruct((NG, D), _f32)),
    )(g, ag, vg)


def _ea_body(e_ref, gie_ref, w_ref, b_ref, out_ref):
    gv = gie_ref[0, 0, :]
    oh = (gv[:, None] == lax.broadcasted_iota(jnp.int32, (1000, NG), 1)).astype(_f32)
    x = jnp.concatenate([e_ref[...], oh], axis=1)
    out_ref[...] = jnp.dot(x, w_ref[...], preferred_element_type=_f32) + b_ref[...]


def _ea(e_prev, gie3, wcat, be):
    return pl.pallas_call(
        _ea_body,
        grid=(N_EDGES // 1000,),
        in_specs=[
            pl.BlockSpec((1000, D), lambda i: (i, 0)),
            pl.BlockSpec((1, 1, 1000), lambda i: (i, 0, 0)),
            pl.BlockSpec((D + NG, D), lambda i: (0, 0)),
            pl.BlockSpec((1, D), lambda i: (0, 0)),
        ],
        out_specs=pl.BlockSpec((1000, D), lambda i: (i, 0)),
        out_shape=jax.ShapeDtypeStruct((N_EDGES, D), _f32),
    )(e_prev, gie3, wcat, be)


def _nprep_body(n_ref, as_ref, ad_ref, ps_ref, pd_ref):
    ps_ref[...] = jnp.dot(n_ref[...], as_ref[...], preferred_element_type=_f32)
    pd_ref[...] = jnp.dot(n_ref[...], ad_ref[...], preferred_element_type=_f32)


def _nprep(n, a_s, a_d):
    return pl.pallas_call(
        _nprep_body,
        grid=(N_NODES // 1000,),
        in_specs=[
            pl.BlockSpec((1000, D), lambda i: (i, 0)),
            pl.BlockSpec((D, D), lambda i: (0, 0)),
            pl.BlockSpec((D, D), lambda i: (0, 0)),
        ],
        out_specs=(pl.BlockSpec((1000, D), lambda i: (i, 0)),
                   pl.BlockSpec((1000, D), lambda i: (i, 0))),
        out_shape=(jax.ShapeDtypeStruct((N_NODES, D), _f32),
                   jax.ShapeDtypeStruct((N_NODES, D), _f32)),
    )(n, a_s, a_d)


def _nv_body(n_ref, a0_ref, a1_ref, gin_ref, vn_ref, ve_ref, tgv_ref, bv_ref,
             out_ref, agg_ref):
    gv = gin_ref[0, 0, :]
    oh = (gv[:, None] == lax.broadcasted_iota(jnp.int32, (1000, NG), 1)).astype(_f32)
    x = jnp.dot(n_ref[...], vn_ref[...], preferred_element_type=_f32)
    x += jnp.dot(a0_ref[...] + a1_ref[...], ve_ref[...], preferred_element_type=_f32)
    x += jnp.dot(oh, tgv_ref[...], preferred_element_type=_f32)
    nn = jnp.maximum(x + bv_ref[...], 0.0)
    out_ref[...] = nn

    @pl.when(pl.program_id(0) == 0)
    def _():
        agg_ref[...] = jnp.zeros_like(agg_ref)

    agg_ref[...] += lax.dot_general(oh, nn, (((0,), (0,)), ((), ())),
                                    preferred_element_type=_f32)


def _nv(n, a0, a1, gin3, vn, ve, tgv, bv):
    return pl.pallas_call(
        _nv_body,
        grid=(N_NODES // 1000,),
        in_specs=[
            pl.BlockSpec((1000, D), lambda i: (i, 0)),
            pl.BlockSpec((1000, D), lambda i: (i, 0)),
            pl.BlockSpec((1000, D), lambda i: (i, 0)),
            pl.BlockSpec((1, 1, 1000), lambda i: (i, 0, 0)),
            pl.BlockSpec((D, D), lambda i: (0, 0)),
            pl.BlockSpec((D, D), lambda i: (0, 0)),
            pl.BlockSpec((NG, D), lambda i: (0, 0)),
            pl.BlockSpec((1, D), lambda i: (0, 0)),
        ],
        out_specs=(pl.BlockSpec((1000, D), lambda i: (i, 0)),
                   pl.BlockSpec((NG, D), lambda i: (0, 0))),
        out_shape=(jax.ShapeDtypeStruct((N_NODES, D), _f32),
                   jax.ShapeDtypeStruct((NG, D), _f32)),
    )(n, a0, a1, gin3, vn, ve, tgv, bv)


def _gv_body(g_ref, an_ref, e0_ref, e1_ref, wg_ref, bg_ref, out_ref):
    x = jnp.concatenate([g_ref[...], an_ref[...], e0_ref[...] + e1_ref[...]],
                        axis=1)
    y = jnp.dot(x, wg_ref[...], preferred_element_type=_f32) + bg_ref[...]
    out_ref[...] = jnp.maximum(y, 0.0)


def _gv(g, aggn, ge0, ge1, wg, bg):
    return pl.pallas_call(
        _gv_body,
        out_shape=jax.ShapeDtypeStruct((NG, D), _f32),
    )(g, aggn, ge0, ge1, wg, bg)


def _sc_mesh():
    return plsc.VectorSubcoreMesh(core_axis_name="c", subcore_axis_name="s",
                                  num_cores=NC)


def _make_gather(B, T):
    nch = B // (NW * CH)

    @functools.partial(
        pl.kernel,
        out_type=jax.ShapeDtypeStruct((B, D), _f32),
        mesh=_sc_mesh(),
        scratch_types=[
            pltpu.VMEM((CH,), jnp.int32),
            pltpu.VMEM((CH, D), _f32),
            pltpu.SemaphoreType.DMA,
        ],
    )
    def k(table_h, idx_h, out_h, idx_v, rows_v, sem):
        wid = lax.axis_index("s") * NC + lax.axis_index("c")
        bpw = nch * CH

        def body(i, carry):
            base = wid * bpw + i * CH
            pltpu.sync_copy(idx_h.at[pl.ds(base, CH)], idx_v)
            pltpu.async_copy(table_h.at[idx_v], rows_v, sem).wait()
            pltpu.sync_copy(rows_v, out_h.at[pl.ds(base, CH)])
            return carry

        lax.fori_loop(0, nch, body, 0)

    return k


def _make_edge_stage():
    @functools.partial(
        pl.kernel,
        out_type=(
            jax.ShapeDtypeStruct((N_EDGES, D), _f32),
            jax.ShapeDtypeStruct((NC, N_NODES, D), _f32),
            jax.ShapeDtypeStruct((NC, NG, D), _f32),
        ),
        mesh=_sc_mesh(),
        scratch_types=[
            pltpu.VMEM((CH, D), _f32),
            pltpu.VMEM((CH, D), _f32),
            pltpu.VMEM((CH, D), _f32),
            pltpu.VMEM((CH, D), _f32),
            pltpu.VMEM((CH,), jnp.int32),
            pltpu.VMEM((2, CH), jnp.int32),
            pltpu.VMEM_SHARED((N_NODES, D), _f32),
            pltpu.VMEM_SHARED((NG, D), _f32),
            pltpu.SemaphoreType.DMA,
            pltpu.SemaphoreType.DMA,
        ],
    )
    def k(ea_h, src_h, dst_h, gie_h, ps_h, pd_h, zeros_h,
          eout_h, aggn_h, aggg_h,
          ea_v, ps_v, pd_v, en_v, sidx_v, scidx_v, aggn_s, aggg_s, sem1, sem2):
        c = lax.axis_index("c")
        s = lax.axis_index("s")
        wid = s * NC + c

        @pl.when(s == 0)
        def _init():
            pltpu.sync_copy(zeros_h, aggn_s)
            pltpu.sync_copy(zeros_h.at[pl.ds(0, NG)], aggg_s)

        plsc.subcore_barrier()

        def body(i, carry):
            base = wid * EPW + i * CH
            ia = [
                pltpu.async_copy(src_h.at[pl.ds(base, CH)], sidx_v, sem1),
                pltpu.async_copy(dst_h.at[pl.ds(base, CH)], scidx_v.at[0],
                                 sem1),
                pltpu.async_copy(gie_h.at[pl.ds(base, CH)], scidx_v.at[1],
                                 sem1),
                pltpu.async_copy(ea_h.at[pl.ds(base, CH)], ea_v, sem1),
            ]
            for cp in ia:
                cp.wait()
            cp1 = pltpu.async_copy(ps_h.at[sidx_v], ps_v, sem1)
            cp2 = pltpu.async_copy(pd_h.at[scidx_v.at[0]], pd_v, sem2)
            cp1.wait()
            cp2.wait()

            def row(r, rc):
                for cb in range(D // L):
                    sl = (r, pl.ds(cb * L, L))
                    en_v[sl] = jnp.maximum(ea_v[sl] + ps_v[sl] + pd_v[sl], 0.0)
                return rc

            lax.fori_loop(0, CH, row, 0)
            ob = [
                pltpu.async_copy(en_v, eout_h.at[pl.ds(base, CH)], sem1),
                pltpu.async_copy(en_v, aggn_s.at[scidx_v.at[0]], sem2,
                                 add=True),
                pltpu.async_copy(en_v, aggg_s.at[scidx_v.at[1]], sem2,
                                 add=True),
            ]
            for cp in ob:
                cp.wait()
            return carry

        lax.fori_loop(0, NCHE, body, 0)
        plsc.subcore_barrier()

        @pl.when(s < 10)
        def _out_n():
            pltpu.sync_copy(aggn_s.at[pl.ds(s * 1000, 1000)],
                            aggn_h.at[c, pl.ds(s * 1000, 1000)])

        @pl.when(s == 0)
        def _out_g():
            pltpu.sync_copy(aggg_s, aggg_h.at[c])

    return k


def _build_patches(image_stack):
    x = jnp.pad(image_stack, ((0, 0), (0, 0), (0, 1), (0, 1)))
    sl = []
    for di in range(3):
        for dj in range(3):
            sl.append(lax.slice(x, (0, 0, di, dj), (NG, 3, di + 223, dj + 223),
                                (1, 1, 2, 2)))
    p = jnp.stack(sl, axis=-1)
    p = p.transpose(0, 2, 3, 1, 4).reshape(NG, 12544, 27)
    p = jnp.pad(p, ((0, 0), (0, 0), (0, 5)))
    return p.reshape(NG * 12544, 32)


def _pad_w(w, rows, cols):
    return jnp.pad(w, ((0, rows - w.shape[0]), (0, cols - w.shape[1])))


def kernel(image_stack, node_categories, edge_categories, edge_connections,
           graph_idx_of_node, graph_idx_of_edge, node_table, edge_table,
           Wconv, bconv, Wfc, bfc, W_e0, b_e0, W_v0, b_v0, W_g0, b_g0,
           W_e1, b_e1, W_v1, b_v1, W_g1, b_g1, W_e2, b_e2, W_v2, b_v2,
           W_g2, b_g2):
    src = edge_connections[0].astype(jnp.int32)
    dst = edge_connections[1].astype(jnp.int32)
    gie = graph_idx_of_edge.astype(jnp.int32)
    gin = graph_idx_of_node.astype(jnp.int32)
    ec = edge_categories.astype(jnp.int32)
    nc_ = node_categories.astype(jnp.int32)

    gie3 = gie.reshape(N_EDGES // 1000, 1, 1000)
    gin3 = gin.reshape(N_NODES // 1000, 1, 1000)
    zeros_big = jnp.zeros((N_NODES, D), _f32)

    W_e2p = _pad_w(W_e2, 512, D)
    b_e2p = jnp.pad(b_e2, (0, D - 1))
    V_n2 = _pad_w(W_v2[:D], D, D)
    V_e2 = _pad_w(W_v2[D:D + 1], D, D)
    V_g2 = _pad_w(W_v2[D + 1:], D, D)
    bv2p = jnp.pad(b_v2, (0, D - 1))
    G_g2 = W_g2[:D]
    G_n2 = _pad_w(W_g2[D:D + 1], D, D)
    G_e2 = _pad_w(W_g2[D + 1:D + 2], D, D)

    layers = []
    for (We, be, Wv, bv, Wg, bg) in ((W_e0, b_e0, W_v0, b_v0, W_g0, b_g0),
                                     (W_e1, b_e1, W_v1, b_v1, W_g1, b_g1)):
        layers.append(dict(
            A_e=We[:D], A_s=We[D:2 * D], A_d=We[2 * D:3 * D], A_g=We[3 * D:],
            be=be.reshape(1, D),
            V_n=Wv[:D], V_e=Wv[D:2 * D], V_g=Wv[2 * D:], bv=bv.reshape(1, D),
            Wgcat=jnp.concatenate([Wg[:D], Wg[D:2 * D], Wg[2 * D:]], axis=0),
            bg=bg.reshape(1, D)))
    layers.append(dict(
        A_e=W_e2p[:D], A_s=W_e2p[D:2 * D], A_d=W_e2p[2 * D:3 * D],
        A_g=W_e2p[3 * D:], be=b_e2p.reshape(1, D),
        V_n=V_n2, V_e=V_e2, V_g=V_g2, bv=bv2p.reshape(1, D),
        Wgcat=jnp.concatenate([G_g2, G_n2, G_e2], axis=0),
        bg=b_g2.reshape(1, D)))

    patches = _build_patches(image_stack)
    wf = jnp.pad(Wconv.reshape(16, 27).T, ((0, 5), (0, 0)))
    pooled = _cnn_pool(patches, wf, bconv.reshape(1, 16)).reshape(NG, 16)
    g, tcomb, tns, tnd, tgv0 = _prep0(
        pooled, Wfc, bfc.reshape(1, D), edge_table, node_table,
        layers[0]["A_e"], layers[0]["A_g"], layers[0]["V_g"],
        layers[0]["A_s"], layers[0]["A_d"], layers[0]["be"])

    gather_e = _make_gather(N_EDGES, 256)
    gather_n = _make_gather(NPAD, 32)
    edge_stage = _make_edge_stage()

    idx0 = ec * NG + gie
    ncp = jnp.pad(nc_, (0, NPAD - N_NODES))
    ea0 = gather_e(tcomb, idx0)
    ps = gather_n(tns, ncp)
    pd = gather_n(tnd, ncp)
    n = gather_n(node_table.astype(_f32), ncp)[:N_NODES]

    e_prev = None
    for i, lay in enumerate(layers):
        if i == 0:
            ea = ea0
            tgv = tgv0
        else:
            tge, tgv = _gprep(g, lay["A_g"], lay["V_g"])
            wcat = jnp.concatenate([lay["A_e"], tge], axis=0)
            ea = _ea(e_prev, gie3, wcat, lay["be"])
            ps, pd = _nprep(n, lay["A_s"], lay["A_d"])
        e_new, aggn, aggg = edge_stage(ea, src, dst, gie, ps, pd, zeros_big)
        n_new, agg_gn = _nv(n, aggn[0], aggn[1], gin3, lay["V_n"], lay["V_e"],
                            tgv, lay["bv"])
        g = _gv(g, agg_gn, aggg[0], aggg[1], lay["Wgcat"], lay["bg"])
        n, e_prev = n_new, e_new

    return (g, n[:, :1], e_prev[:, :1])
